# Initial kernel scaffold; baseline (speedup 1.0000x reference)
import jax
import jax.numpy as jnp
from jax.experimental import pallas as pl


def kernel(X, hyperedge_index, theta_v, reference_pts, weight):
    raise NotImplementedError("write your pallas kernel here")


# speedup vs baseline: 1.9342x; 1.9342x over previous
import functools

import jax
import jax.numpy as jnp
from jax import lax
from jax.experimental import pallas as pl
from jax.experimental.pallas import tpu as pltpu
from jax.experimental.pallas import tpu_sc as plsc

N = 32768
C = 64
P = 32
M = 128
S = 16
_C0 = 0.99998 / 0.99999 / (M - 1)


def _ce_pass(key, li, block, dist):
    up = pltpu.roll(key, N - dist, 1)
    dn = pltpu.roll(key, dist, 1)
    h = (li & dist) == 0
    asc = (li & block) == 0
    partner = jnp.where(h, up, dn)
    take_min = h == asc
    return jnp.where(take_min, jnp.minimum(key, partner),
                     jnp.maximum(key, partner))


def _prep_body(xt_ref, tv_ref, seg_ref, key_out, fmin_out, scale_out,
               idxg_out, idxm_out, frac_out):
    tv = tv_ref[...]
    w = tv / jnp.sqrt(jnp.sum(tv * tv, axis=1, keepdims=True))
    xs = lax.dot_general(w, xt_ref[...], (((1,), (0,)), ((), ())),
                         preferred_element_type=jnp.float32)
    fmin = jnp.min(xs, axis=1, keepdims=True)
    fmax = jnp.max(xs, axis=1, keepdims=True)
    scale = fmax - fmin + 1e-12
    fmin_out[...] = fmin
    scale_out[...] = scale

    seg2d = seg_ref[...]
    cnt = [jnp.sum((seg2d == e).astype(jnp.int32)) for e in range(S)]
    ptrs = []
    run = jnp.int32(0)
    for e in range(S):
        ptrs.append(run)
        run = run + cnt[e]

    li = lax.broadcasted_iota(jnp.int32, (1, N), 1)
    segb = jnp.zeros((1, N), jnp.int32)
    for e in range(1, S):
        segb = segb + (li >= ptrs[e]).astype(jnp.int32)

    key = (xs - fmin) / scale * 0.998 + 0.001 + segb.astype(jnp.float32)

    def stage(s, k):
        block = jnp.int32(2) << s

        def cond(c):
            return c[0] >= 1

        def body(c):
            dist, kk = c
            return (dist // 2, _ce_pass(kk, li, block, dist))

        _, k = lax.while_loop(cond, body, (block // 2, k))
        return k

    key_out[...] = lax.fori_loop(0, 15, stage, key)

    jj = lax.broadcasted_iota(jnp.int32, (1, M), 1).astype(jnp.float32)
    base = jj * _C0
    for e in range(S):
        dreal = cnt[e]
        daug = jnp.where(dreal == 1, 2, dreal)
        t = base * (daug - 1).astype(jnp.float32)
        ilo = jnp.floor(t)
        fr = t - ilo
        ilo = ilo.astype(jnp.int32)
        cap = dreal - 1
        lo = jnp.maximum(ptrs[e] + jnp.minimum(ilo, cap), 0)
        hi = jnp.maximum(ptrs[e] + jnp.minimum(ilo + 1, cap), 0)
        idxg_out[pl.ds(e, 1), :] = lo >> 2
        idxg_out[pl.ds(S + e, 1), :] = hi >> 2
        idxm_out[pl.ds(e, 1), :] = lo & 3
        idxm_out[pl.ds(S + e, 1), :] = hi & 3
        frac_out[pl.ds(e, 1), :] = fr


def _finish_body(v_ref, qm_ref, frac_ref, fmin_ref, scale_ref, wt_ref,
                 ref_ref, out_ref):
    rows = v_ref[...]
    qm = qm_ref[...]
    v = jnp.zeros((2 * S * M, P), jnp.float32)
    for q in range(4):
        v = v + jnp.where(qm == q, rows[:, q * P:(q + 1) * P], 0.0)
    segf = jnp.floor(v)
    vals = (v - segf - 0.001) * (1.0 / 0.998) * scale_ref[...] + fmin_ref[...]
    vlo = vals[:S * M]
    vhi = vals[S * M:]
    f = frac_ref[...]
    yq = vlo + f * (vhi - vlo)
    wt = wt_ref[...]
    refp = ref_ref[...]
    c = jnp.mean(wt * refp, axis=0, keepdims=True)
    s = jnp.sum(yq.reshape(S, M, P) * wt[None, :, :], axis=1)
    out_ref[...] = c - s * (1.0 / M)


@functools.cache
def _make_sc_gather():
    info = plsc.get_sparse_core_info()
    nw = info.num_cores * info.num_subcores
    b = 2 * S * M
    bpw = b // nw
    mesh = plsc.VectorSubcoreMesh(core_axis_name="c", subcore_axis_name="s")

    @functools.partial(
        pl.kernel, mesh=mesh,
        out_type=jax.ShapeDtypeStruct((b, 4 * P), jnp.float32),
        scratch_types=[
            pltpu.VMEM((bpw,), jnp.int32),
            pltpu.VMEM((bpw, 4 * P), jnp.float32),
            pltpu.SemaphoreType.DMA,
        ],
    )
    def sc_gather(idx_hbm, table_hbm, out_hbm, idx_v, rows_v, sem):
        wid = lax.axis_index("s") * info.num_cores + lax.axis_index("c")
        base = wid * bpw
        pltpu.sync_copy(idx_hbm.at[pl.ds(base, bpw)], idx_v)
        pltpu.async_copy(table_hbm.at[idx_v], rows_v, sem).wait()
        pltpu.sync_copy(rows_v, out_hbm.at[pl.ds(base, bpw)])

    return sc_gather


def kernel(X, hyperedge_index, theta_v, reference_pts, weight):
    seg2d = hyperedge_index.astype(jnp.int32).reshape(N // M, M)
    key_sorted, fmin, scale, idxg2d, idxm2d, frac2d = pl.pallas_call(
        _prep_body,
        out_shape=[
            jax.ShapeDtypeStruct((P, N), jnp.float32),
            jax.ShapeDtypeStruct((P, 1), jnp.float32),
            jax.ShapeDtypeStruct((P, 1), jnp.float32),
            jax.ShapeDtypeStruct((2 * S, M), jnp.int32),
            jax.ShapeDtypeStruct((2 * S, M), jnp.int32),
            jax.ShapeDtypeStruct((S, M), jnp.float32),
        ],
    )(X.T, theta_v, seg2d)

    table = key_sorted.T.reshape(N // 4, 4 * P)
    gathered = _make_sc_gather()(idxg2d.reshape(2 * S * M), table)

    out = pl.pallas_call(
        _finish_body,
        out_shape=jax.ShapeDtypeStruct((S, P), jnp.float32),
    )(gathered, idxm2d.reshape(2 * S * M, 1), frac2d.reshape(S * M, 1),
      fmin.reshape(1, P), scale.reshape(1, P), weight.T, reference_pts)
    edges = jnp.arange(S, dtype=hyperedge_index.dtype)
    return (out, edges)
